# Initial kernel scaffold; baseline (speedup 1.0000x reference)
import jax
import jax.numpy as jnp
from jax.experimental import pallas as pl


def kernel(query, key, value, Wq, bq, Wk, bk, Wv, bv, Wo, bo, R):
    raise NotImplementedError("write your pallas kernel here")


# speedup vs baseline: 2.6420x; 2.6420x over previous
import functools

import jax
import jax.numpy as jnp
from jax.experimental import pallas as pl

EMBED = 1024
HEADS = 16
HEAD_DIM = EMBED // HEADS
N_HASHES = 32
PROJ = 4
SEQ = 2048


def _proj_kernel(x_ref, w_ref, b_ref, o_ref):
    o_ref[0] = (
        jnp.dot(x_ref[0], w_ref[0], preferred_element_type=jnp.float32) + b_ref[0]
    )


def _sig_kernel(h_ref, r_ref, o_ref):
    a = jnp.dot(h_ref[0], r_ref[...], preferred_element_type=jnp.float32)
    c0 = a[:, 0 * N_HASHES : 1 * N_HASHES]
    c1 = a[:, 1 * N_HASHES : 2 * N_HASHES]
    c2 = a[:, 2 * N_HASHES : 3 * N_HASHES]
    c3 = a[:, 3 * N_HASHES : 4 * N_HASHES]
    best = c0
    idx = jnp.zeros_like(c0, dtype=jnp.int32)
    for j, c in ((1, c1), (2, c2), (3, c3)):
        upd = c > best
        idx = jnp.where(upd, j, idx)
        best = jnp.where(upd, c, best)
    sig = jnp.concatenate(
        [(idx == j).astype(jnp.float32) for j in range(4)], axis=1
    )
    o_ref[0] = sig


def _attn_kernel(q_ref, k_ref, v_ref, sq_ref, sk_ref, o_ref):
    dims = (((1,), (1,)), ((), ()))
    s = jax.lax.dot_general(
        q_ref[0], k_ref[0], dims, preferred_element_type=jnp.float32
    ) * (1.0 / (HEAD_DIM ** 0.5))
    cnt = jax.lax.dot_general(
        sq_ref[0], sk_ref[0], dims, preferred_element_type=jnp.float32
    )
    s = jnp.where(cnt > 0.5, s, jnp.float32(-1e9))
    m = jnp.max(s, axis=1, keepdims=True)
    p = jnp.exp(s - m)
    denom = jnp.sum(p, axis=1, keepdims=True)
    o_ref[0] = jnp.dot(p, v_ref[0], preferred_element_type=jnp.float32) / denom


def _out_kernel(x_ref, w_ref, b_ref, o_ref):
    o_ref[...] = (
        jnp.dot(x_ref[...], w_ref[...], preferred_element_type=jnp.float32)
        + b_ref[...]
    )


@jax.jit
def kernel(query, key, value, Wq, bq, Wk, bk, Wv, bv, Wo, bo, R):
    B = query.shape[1]
    assert B == 1
    xq = query[:, 0, :]
    xk = key[:, 0, :]
    xv = value[:, 0, :]

    xs = jnp.stack([xq, xk, xv])
    ws = jnp.stack([Wq.T, Wk.T, Wv.T])
    bs = jnp.stack([bq, bk, bv])[:, None, :]

    qkv = pl.pallas_call(
        _proj_kernel,
        grid=(3,),
        in_specs=[
            pl.BlockSpec((1, SEQ, EMBED), lambda i: (i, 0, 0)),
            pl.BlockSpec((1, EMBED, EMBED), lambda i: (i, 0, 0)),
            pl.BlockSpec((1, 1, EMBED), lambda i: (i, 0, 0)),
        ],
        out_specs=pl.BlockSpec((1, SEQ, EMBED), lambda i: (i, 0, 0)),
        out_shape=jax.ShapeDtypeStruct((3, SEQ, EMBED), jnp.float32),
    )(xs, ws, bs)

    heads = qkv.reshape(3, SEQ, HEADS, HEAD_DIM).transpose(0, 2, 1, 3)
    qh, kh, vh = heads[0], heads[1], heads[2]

    r2 = R[0].transpose(0, 2, 1).reshape(HEAD_DIM, 2 * N_HASHES)
    rfull = jnp.concatenate([r2, -r2], axis=1)

    qk_stack = jnp.concatenate([qh, kh], axis=0)
    SIG = 4 * N_HASHES
    sigs = pl.pallas_call(
        _sig_kernel,
        grid=(2 * HEADS,),
        in_specs=[
            pl.BlockSpec((1, SEQ, HEAD_DIM), lambda i: (i, 0, 0)),
            pl.BlockSpec((HEAD_DIM, SIG), lambda i: (0, 0)),
        ],
        out_specs=pl.BlockSpec((1, SEQ, SIG), lambda i: (i, 0, 0)),
        out_shape=jax.ShapeDtypeStruct((2 * HEADS, SEQ, SIG), jnp.float32),
    )(qk_stack, rfull)
    sq, sk = sigs[:HEADS], sigs[HEADS:]

    BQ = 512
    NQ = SEQ // BQ
    att = pl.pallas_call(
        _attn_kernel,
        grid=(HEADS, NQ),
        in_specs=[
            pl.BlockSpec((1, BQ, HEAD_DIM), lambda h, i: (h, i, 0)),
            pl.BlockSpec((1, SEQ, HEAD_DIM), lambda h, i: (h, 0, 0)),
            pl.BlockSpec((1, SEQ, HEAD_DIM), lambda h, i: (h, 0, 0)),
            pl.BlockSpec((1, BQ, SIG), lambda h, i: (h, i, 0)),
            pl.BlockSpec((1, SEQ, SIG), lambda h, i: (h, 0, 0)),
        ],
        out_specs=pl.BlockSpec((1, BQ, HEAD_DIM), lambda h, i: (h, i, 0)),
        out_shape=jax.ShapeDtypeStruct((HEADS, SEQ, HEAD_DIM), jnp.float32),
    )(qh, kh, vh, sq, sk)

    merged = att.transpose(1, 0, 2).reshape(SEQ, EMBED)

    out = pl.pallas_call(
        _out_kernel,
        in_specs=[
            pl.BlockSpec((SEQ, EMBED), lambda: (0, 0)),
            pl.BlockSpec((EMBED, EMBED), lambda: (0, 0)),
            pl.BlockSpec((1, EMBED), lambda: (0, 0)),
        ],
        out_specs=pl.BlockSpec((SEQ, EMBED), lambda: (0, 0)),
        out_shape=jax.ShapeDtypeStruct((SEQ, EMBED), jnp.float32),
    )(merged, Wo.T, bo[None, :])

    return out[:, None, :]


# speedup vs baseline: 3.1236x; 1.1823x over previous
import jax
import jax.numpy as jnp
from jax.experimental import pallas as pl

EMBED = 1024
HEADS = 16
HEAD_DIM = EMBED // HEADS
N_HASHES = 32
SIG = 4 * N_HASHES
SEQ = 2048


def _qk_kernel(x_ref, w_ref, b_ref, r_ref, h_ref, s_ref):
    h = (
        jnp.dot(x_ref[0], w_ref[0, 0], preferred_element_type=jnp.float32)
        + b_ref[0, 0]
    )
    h_ref[0, 0] = h
    a = jnp.dot(h, r_ref[...], preferred_element_type=jnp.float32)
    c0 = a[:, 0 * N_HASHES : 1 * N_HASHES]
    c1 = a[:, 1 * N_HASHES : 2 * N_HASHES]
    c2 = a[:, 2 * N_HASHES : 3 * N_HASHES]
    c3 = a[:, 3 * N_HASHES : 4 * N_HASHES]
    best = c0
    idx = jnp.zeros_like(c0, dtype=jnp.int32)
    for j, c in ((1, c1), (2, c2), (3, c3)):
        upd = c > best
        idx = jnp.where(upd, j, idx)
        best = jnp.where(upd, c, best)
    s_ref[0, 0] = jnp.concatenate(
        [(idx == j).astype(jnp.bfloat16) for j in range(4)], axis=1
    )


def _v_kernel(x_ref, w_ref, b_ref, o_ref):
    o_ref[0] = (
        jnp.dot(x_ref[...], w_ref[0], preferred_element_type=jnp.float32)
        + b_ref[0]
    )


def _attn_kernel(q_ref, k_ref, v_ref, sq_ref, sk_ref, wo_ref, bo_ref, o_ref):
    dims = (((1,), (1,)), ((), ()))
    s = jax.lax.dot_general(
        q_ref[0, 0], k_ref[0, 0], dims, preferred_element_type=jnp.float32
    ) * (1.0 / (HEAD_DIM ** 0.5))
    cnt = jax.lax.dot_general(
        sq_ref[0, 0], sk_ref[0, 0], dims, preferred_element_type=jnp.float32
    )
    s = jnp.where(cnt > 0.5, s, jnp.float32(-1e9))
    m = jnp.max(s, axis=1, keepdims=True)
    p = jnp.exp(s - m)
    denom = jnp.sum(p, axis=1, keepdims=True)
    oh = jnp.dot(p, v_ref[0], preferred_element_type=jnp.float32) / denom
    contrib = jnp.dot(oh, wo_ref[0], preferred_element_type=jnp.float32)

    @pl.when(pl.program_id(1) == 0)
    def _init():
        o_ref[...] = contrib + bo_ref[...]

    @pl.when(pl.program_id(1) != 0)
    def _acc():
        o_ref[...] += contrib


@jax.jit
def kernel(query, key, value, Wq, bq, Wk, bk, Wv, bv, Wo, bo, R):
    assert query.shape[1] == 1
    xq = query[:, 0, :]
    xk = key[:, 0, :]
    xv = value[:, 0, :]

    wqk = jnp.stack([Wq.T, Wk.T]).reshape(2, EMBED, HEADS, HEAD_DIM)
    wqk = wqk.transpose(0, 2, 1, 3)
    bqk = jnp.stack([bq, bk]).reshape(2, 1, HEADS, HEAD_DIM)
    bqk = bqk.transpose(0, 2, 1, 3)
    xqk = jnp.stack([xq, xk])

    r2 = R[0].transpose(0, 2, 1).reshape(HEAD_DIM, 2 * N_HASHES)
    rfull = jnp.concatenate([r2, -r2], axis=1)

    heads, sigs = pl.pallas_call(
        _qk_kernel,
        grid=(2, HEADS),
        in_specs=[
            pl.BlockSpec((1, SEQ, EMBED), lambda i, h: (i, 0, 0)),
            pl.BlockSpec((1, 1, EMBED, HEAD_DIM), lambda i, h: (i, h, 0, 0)),
            pl.BlockSpec((1, 1, 1, HEAD_DIM), lambda i, h: (i, h, 0, 0)),
            pl.BlockSpec((HEAD_DIM, SIG), lambda i, h: (0, 0)),
        ],
        out_specs=[
            pl.BlockSpec((1, 1, SEQ, HEAD_DIM), lambda i, h: (i, h, 0, 0)),
            pl.BlockSpec((1, 1, SEQ, SIG), lambda i, h: (i, h, 0, 0)),
        ],
        out_shape=[
            jax.ShapeDtypeStruct((2, HEADS, SEQ, HEAD_DIM), jnp.float32),
            jax.ShapeDtypeStruct((2, HEADS, SEQ, SIG), jnp.bfloat16),
        ],
    )(xqk, wqk, bqk, rfull)

    wv = Wv.T.reshape(EMBED, HEADS, HEAD_DIM).transpose(1, 0, 2)
    bvh = bv.reshape(1, HEADS, HEAD_DIM).transpose(1, 0, 2)
    vh = pl.pallas_call(
        _v_kernel,
        grid=(HEADS,),
        in_specs=[
            pl.BlockSpec((SEQ, EMBED), lambda h: (0, 0)),
            pl.BlockSpec((1, EMBED, HEAD_DIM), lambda h: (h, 0, 0)),
            pl.BlockSpec((1, 1, HEAD_DIM), lambda h: (h, 0, 0)),
        ],
        out_specs=pl.BlockSpec((1, SEQ, HEAD_DIM), lambda h: (h, 0, 0)),
        out_shape=jax.ShapeDtypeStruct((HEADS, SEQ, HEAD_DIM), jnp.float32),
    )(xv, wv, bvh)

    woh = Wo.T.reshape(HEADS, HEAD_DIM, EMBED)

    BQ = 512
    NQ = SEQ // BQ
    out = pl.pallas_call(
        _attn_kernel,
        grid=(NQ, HEADS),
        in_specs=[
            pl.BlockSpec((1, 1, BQ, HEAD_DIM), lambda i, h: (0, h, i, 0)),
            pl.BlockSpec((1, 1, SEQ, HEAD_DIM), lambda i, h: (1, h, 0, 0)),
            pl.BlockSpec((1, SEQ, HEAD_DIM), lambda i, h: (h, 0, 0)),
            pl.BlockSpec((1, 1, BQ, SIG), lambda i, h: (0, h, i, 0)),
            pl.BlockSpec((1, 1, SEQ, SIG), lambda i, h: (1, h, 0, 0)),
            pl.BlockSpec((1, HEAD_DIM, EMBED), lambda i, h: (h, 0, 0)),
            pl.BlockSpec((1, EMBED), lambda i, h: (0, 0)),
        ],
        out_specs=pl.BlockSpec((BQ, EMBED), lambda i, h: (i, 0)),
        out_shape=jax.ShapeDtypeStruct((SEQ, EMBED), jnp.float32),
    )(heads, heads, vh, sigs, sigs, woh, bo[None, :])

    return out[:, None, :]


# speedup vs baseline: 3.7278x; 1.1934x over previous
import jax
import jax.numpy as jnp
from jax.experimental import pallas as pl

EMBED = 1024
HEADS = 16
HEAD_DIM = EMBED // HEADS
N_HASHES = 32
SIG = 4 * N_HASHES
SEQ = 2048
CDIMS = (((1,), (1,)), ((), ()))


def _proj_sig_kernel(x_ref, w_ref, b_ref, r_ref, h_ref, s_ref):
    h = (
        jax.lax.dot_general(
            x_ref[:, 0, :], w_ref[...], CDIMS, preferred_element_type=jnp.float32
        )
        + b_ref[0]
    )
    h_ref[0] = h
    a = jnp.dot(h, r_ref[...], preferred_element_type=jnp.float32)
    c0 = a[:, 0 * N_HASHES : 1 * N_HASHES]
    c1 = a[:, 1 * N_HASHES : 2 * N_HASHES]
    c2 = a[:, 2 * N_HASHES : 3 * N_HASHES]
    c3 = a[:, 3 * N_HASHES : 4 * N_HASHES]
    best = c0
    idx = jnp.zeros_like(c0, dtype=jnp.int32)
    for j, c in ((1, c1), (2, c2), (3, c3)):
        upd = c > best
        idx = jnp.where(upd, j, idx)
        best = jnp.where(upd, c, best)
    s_ref[0] = jnp.concatenate(
        [(idx == j).astype(jnp.bfloat16) for j in range(4)], axis=1
    )


def _v_kernel(x_ref, w_ref, b_ref, o_ref, m_ref):
    o = (
        jax.lax.dot_general(
            x_ref[:, 0, :], w_ref[...], CDIMS, preferred_element_type=jnp.float32
        )
        + b_ref[0]
    )
    o_ref[0] = o
    m_ref[0] = jnp.mean(o, axis=0, keepdims=True)


def _attn_kernel(q_ref, k_ref, v_ref, vm_ref, sq_ref, sk_ref, wo_ref, bo_ref, o_ref):
    q = q_ref[0] * (1.0 / (HEAD_DIM ** 0.5))
    s = jax.lax.dot_general(q, k_ref[0], CDIMS, preferred_element_type=jnp.float32)
    cnt = jax.lax.dot_general(
        sq_ref[0], sk_ref[0], CDIMS, preferred_element_type=jnp.float32
    )
    p = jnp.where(cnt > 0.5, jnp.exp(s), 0.0)
    denom = jnp.sum(p, axis=1, keepdims=True)
    pv = jnp.dot(p, v_ref[0], preferred_element_type=jnp.float32)
    dead = denom == 0.0
    oh = pv / jnp.where(dead, 1.0, denom)
    oh = jnp.where(dead, vm_ref[0], oh)
    contrib = jnp.dot(oh, wo_ref[0], preferred_element_type=jnp.float32)

    @pl.when(pl.program_id(1) == 0)
    def _init():
        o_ref[...] = contrib + bo_ref[...]

    @pl.when(pl.program_id(1) != 0)
    def _acc():
        o_ref[...] += contrib


def _proj_sig(x, W, b, rfull):
    return pl.pallas_call(
        _proj_sig_kernel,
        grid=(HEADS,),
        in_specs=[
            pl.BlockSpec((SEQ, 1, EMBED), lambda h: (0, 0, 0)),
            pl.BlockSpec((HEAD_DIM, EMBED), lambda h: (h, 0)),
            pl.BlockSpec((1, 1, HEAD_DIM), lambda h: (h, 0, 0)),
            pl.BlockSpec((HEAD_DIM, SIG), lambda h: (0, 0)),
        ],
        out_specs=[
            pl.BlockSpec((1, SEQ, HEAD_DIM), lambda h: (h, 0, 0)),
            pl.BlockSpec((1, SEQ, SIG), lambda h: (h, 0, 0)),
        ],
        out_shape=[
            jax.ShapeDtypeStruct((HEADS, SEQ, HEAD_DIM), jnp.float32),
            jax.ShapeDtypeStruct((HEADS, SEQ, SIG), jnp.bfloat16),
        ],
    )(x, W, b.reshape(HEADS, 1, HEAD_DIM), rfull)


@jax.jit
def kernel(query, key, value, Wq, bq, Wk, bk, Wv, bv, Wo, bo, R):
    assert query.shape[1] == 1

    r2 = R[0].transpose(0, 2, 1).reshape(HEAD_DIM, 2 * N_HASHES)
    rfull = jnp.concatenate([r2, -r2], axis=1)

    qh, sq = _proj_sig(query, Wq, bq, rfull)
    kh, sk = _proj_sig(key, Wk, bk, rfull)

    vh, vmean = pl.pallas_call(
        _v_kernel,
        grid=(HEADS,),
        in_specs=[
            pl.BlockSpec((SEQ, 1, EMBED), lambda h: (0, 0, 0)),
            pl.BlockSpec((HEAD_DIM, EMBED), lambda h: (h, 0)),
            pl.BlockSpec((1, 1, HEAD_DIM), lambda h: (h, 0, 0)),
        ],
        out_specs=[
            pl.BlockSpec((1, SEQ, HEAD_DIM), lambda h: (h, 0, 0)),
            pl.BlockSpec((1, 1, HEAD_DIM), lambda h: (h, 0, 0)),
        ],
        out_shape=[
            jax.ShapeDtypeStruct((HEADS, SEQ, HEAD_DIM), jnp.float32),
            jax.ShapeDtypeStruct((HEADS, 1, HEAD_DIM), jnp.float32),
        ],
    )(value, Wv, bv.reshape(HEADS, 1, HEAD_DIM))

    BQ = 512
    NQ = SEQ // BQ
    out = pl.pallas_call(
        _attn_kernel,
        grid=(NQ, HEADS),
        in_specs=[
            pl.BlockSpec((1, BQ, HEAD_DIM), lambda i, h: (h, i, 0)),
            pl.BlockSpec((1, SEQ, HEAD_DIM), lambda i, h: (h, 0, 0)),
            pl.BlockSpec((1, SEQ, HEAD_DIM), lambda i, h: (h, 0, 0)),
            pl.BlockSpec((1, 1, HEAD_DIM), lambda i, h: (h, 0, 0)),
            pl.BlockSpec((1, BQ, SIG), lambda i, h: (h, i, 0)),
            pl.BlockSpec((1, SEQ, SIG), lambda i, h: (h, 0, 0)),
            pl.BlockSpec((1, HEAD_DIM, EMBED), lambda i, h: (h, 0, 0)),
            pl.BlockSpec((1, EMBED), lambda i, h: (0, 0)),
        ],
        out_specs=pl.BlockSpec((BQ, EMBED), lambda i, h: (i, 0)),
        out_shape=jax.ShapeDtypeStruct((SEQ, EMBED), jnp.float32),
    )(qh, kh, vh, vmean, sq, sk, Wo.T.reshape(HEADS, HEAD_DIM, EMBED), bo[None, :])

    return out[:, None, :]


# speedup vs baseline: 3.8385x; 1.0297x over previous
import functools

import jax
import jax.numpy as jnp
from jax.experimental import pallas as pl

EMBED = 1024
HEADS = 16
HEAD_DIM = EMBED // HEADS
N_HASHES = 32
SIG = 4 * N_HASHES
SEQ = 2048
CDIMS = (((1,), (1,)), ((), ()))


def _proj_sig_kernel(x_ref, w_ref, b_ref, r_ref, h_ref, s_ref, *, scale):
    h = (
        jax.lax.dot_general(
            x_ref[:, 0, :], w_ref[...], CDIMS, preferred_element_type=jnp.float32
        )
        + b_ref[0]
    )
    h_ref[0] = (h * scale).astype(jnp.bfloat16)
    a = jnp.dot(h, r_ref[...], preferred_element_type=jnp.float32)
    c0 = a[:, 0 * N_HASHES : 1 * N_HASHES]
    c1 = a[:, 1 * N_HASHES : 2 * N_HASHES]
    c2 = a[:, 2 * N_HASHES : 3 * N_HASHES]
    c3 = a[:, 3 * N_HASHES : 4 * N_HASHES]
    best = c0
    idx = jnp.zeros_like(c0, dtype=jnp.int32)
    for j, c in ((1, c1), (2, c2), (3, c3)):
        upd = c > best
        idx = jnp.where(upd, j, idx)
        best = jnp.where(upd, c, best)
    s_ref[0] = jnp.concatenate(
        [(idx == j).astype(jnp.bfloat16) for j in range(4)], axis=1
    )


def _v_kernel(x_ref, w_ref, b_ref, o_ref, m_ref):
    o = (
        jax.lax.dot_general(
            x_ref[:, 0, :], w_ref[...], CDIMS, preferred_element_type=jnp.float32
        )
        + b_ref[0]
    )
    o_ref[0] = o.astype(jnp.bfloat16)
    m_ref[0] = jnp.mean(o, axis=0, keepdims=True)


def _attn_kernel(q_ref, k_ref, v_ref, vm_ref, sq_ref, sk_ref, wo_ref, bo_ref, o_ref):
    s = jax.lax.dot_general(
        q_ref[0], k_ref[0], CDIMS, preferred_element_type=jnp.float32
    )
    cnt = jax.lax.dot_general(
        sq_ref[0], sk_ref[0], CDIMS, preferred_element_type=jnp.float32
    )
    p = jnp.where(cnt > 0.5, jnp.exp(s), 0.0)
    denom = jnp.sum(p, axis=1, keepdims=True)
    pv = jnp.dot(
        p.astype(jnp.bfloat16), v_ref[0], preferred_element_type=jnp.float32
    )
    dead = denom == 0.0
    oh = pv / jnp.where(dead, 1.0, denom)
    oh = jnp.where(dead, vm_ref[0], oh)
    contrib = jnp.dot(oh, wo_ref[0], preferred_element_type=jnp.float32)

    @pl.when(pl.program_id(1) == 0)
    def _init():
        o_ref[...] = contrib + bo_ref[...]

    @pl.when(pl.program_id(1) != 0)
    def _acc():
        o_ref[...] += contrib


def _proj_sig(x, W, b, rfull, scale):
    return pl.pallas_call(
        functools.partial(_proj_sig_kernel, scale=scale),
        grid=(HEADS,),
        in_specs=[
            pl.BlockSpec((SEQ, 1, EMBED), lambda h: (0, 0, 0)),
            pl.BlockSpec((HEAD_DIM, EMBED), lambda h: (h, 0)),
            pl.BlockSpec((1, 1, HEAD_DIM), lambda h: (h, 0, 0)),
            pl.BlockSpec((HEAD_DIM, SIG), lambda h: (0, 0)),
        ],
        out_specs=[
            pl.BlockSpec((1, SEQ, HEAD_DIM), lambda h: (h, 0, 0)),
            pl.BlockSpec((1, SEQ, SIG), lambda h: (h, 0, 0)),
        ],
        out_shape=[
            jax.ShapeDtypeStruct((HEADS, SEQ, HEAD_DIM), jnp.bfloat16),
            jax.ShapeDtypeStruct((HEADS, SEQ, SIG), jnp.bfloat16),
        ],
    )(x, W, b.reshape(HEADS, 1, HEAD_DIM), rfull)


@jax.jit
def kernel(query, key, value, Wq, bq, Wk, bk, Wv, bv, Wo, bo, R):
    assert query.shape[1] == 1

    r2 = R[0].transpose(0, 2, 1).reshape(HEAD_DIM, 2 * N_HASHES)
    rfull = jnp.concatenate([r2, -r2], axis=1)

    qh, sq = _proj_sig(query, Wq, bq, rfull, 1.0 / (HEAD_DIM ** 0.5))
    kh, sk = _proj_sig(key, Wk, bk, rfull, 1.0)

    vh, vmean = pl.pallas_call(
        _v_kernel,
        grid=(HEADS,),
        in_specs=[
            pl.BlockSpec((SEQ, 1, EMBED), lambda h: (0, 0, 0)),
            pl.BlockSpec((HEAD_DIM, EMBED), lambda h: (h, 0)),
            pl.BlockSpec((1, 1, HEAD_DIM), lambda h: (h, 0, 0)),
        ],
        out_specs=[
            pl.BlockSpec((1, SEQ, HEAD_DIM), lambda h: (h, 0, 0)),
            pl.BlockSpec((1, 1, HEAD_DIM), lambda h: (h, 0, 0)),
        ],
        out_shape=[
            jax.ShapeDtypeStruct((HEADS, SEQ, HEAD_DIM), jnp.bfloat16),
            jax.ShapeDtypeStruct((HEADS, 1, HEAD_DIM), jnp.float32),
        ],
    )(value, Wv, bv.reshape(HEADS, 1, HEAD_DIM))

    BQ = 512
    NQ = SEQ // BQ
    out = pl.pallas_call(
        _attn_kernel,
        grid=(NQ, HEADS),
        in_specs=[
            pl.BlockSpec((1, BQ, HEAD_DIM), lambda i, h: (h, i, 0)),
            pl.BlockSpec((1, SEQ, HEAD_DIM), lambda i, h: (h, 0, 0)),
            pl.BlockSpec((1, SEQ, HEAD_DIM), lambda i, h: (h, 0, 0)),
            pl.BlockSpec((1, 1, HEAD_DIM), lambda i, h: (h, 0, 0)),
            pl.BlockSpec((1, BQ, SIG), lambda i, h: (h, i, 0)),
            pl.BlockSpec((1, SEQ, SIG), lambda i, h: (h, 0, 0)),
            pl.BlockSpec((1, HEAD_DIM, EMBED), lambda i, h: (h, 0, 0)),
            pl.BlockSpec((1, EMBED), lambda i, h: (0, 0)),
        ],
        out_specs=pl.BlockSpec((BQ, EMBED), lambda i, h: (i, 0)),
        out_shape=jax.ShapeDtypeStruct((SEQ, EMBED), jnp.float32),
    )(qh, kh, vh, vmean, sq, sk, Wo.T.reshape(HEADS, HEAD_DIM, EMBED), bo[None, :])

    return out[:, None, :]


# speedup vs baseline: 4.3061x; 1.1218x over previous
import jax
import jax.numpy as jnp
from jax.experimental import pallas as pl

EMBED = 1024
HEADS = 16
HEAD_DIM = EMBED // HEADS
N_HASHES = 32
SIG = 4 * N_HASHES
SEQ = 2048
CDIMS = (((1,), (1,)), ((), ()))


def _signature(h, r):
    a = jnp.dot(h, r, preferred_element_type=jnp.float32)
    c0 = a[:, 0 * N_HASHES : 1 * N_HASHES]
    c1 = a[:, 1 * N_HASHES : 2 * N_HASHES]
    c2 = a[:, 2 * N_HASHES : 3 * N_HASHES]
    c3 = a[:, 3 * N_HASHES : 4 * N_HASHES]
    best = c0
    idx = jnp.zeros_like(c0, dtype=jnp.int32)
    for j, c in ((1, c1), (2, c2), (3, c3)):
        upd = c > best
        idx = jnp.where(upd, j, idx)
        best = jnp.where(upd, c, best)
    return jnp.concatenate(
        [(idx == j).astype(jnp.bfloat16) for j in range(4)], axis=1
    )


def _proj(x_ref, w_ref, b_ref):
    return (
        jax.lax.dot_general(
            x_ref[:, 0, :], w_ref[...], CDIMS, preferred_element_type=jnp.float32
        )
        + b_ref[0]
    )


def _kv_kernel(
    xk_ref, wk_ref, bk_ref, r_ref, xv_ref, wv_ref, bv_ref, k_ref, sk_ref, v_ref, m_ref
):
    k = _proj(xk_ref, wk_ref, bk_ref)
    k_ref[0] = k.astype(jnp.bfloat16)
    sk_ref[0] = _signature(k, r_ref[...])
    v = _proj(xv_ref, wv_ref, bv_ref)
    unit = (
        jax.lax.broadcasted_iota(jnp.int32, (SEQ, 128 - HEAD_DIM), 1) == 0
    ).astype(jnp.float32)
    v_ref[0] = jnp.concatenate([v, unit], axis=1).astype(jnp.bfloat16)
    m_ref[0] = jnp.mean(v, axis=0, keepdims=True)


def _attn_kernel(
    xq_ref, wq_ref, bq_ref, r_ref, k_ref, v_ref, vm_ref, sk_ref, wo_ref, bo_ref, o_ref
):
    q = _proj(xq_ref, wq_ref, bq_ref) * (1.0 / (HEAD_DIM ** 0.5))
    sq = _signature(q, r_ref[...])
    s = jax.lax.dot_general(
        q.astype(jnp.bfloat16), k_ref[0], CDIMS, preferred_element_type=jnp.float32
    )
    cnt = jax.lax.dot_general(sq, sk_ref[0], CDIMS, preferred_element_type=jnp.float32)
    p = jnp.where(cnt > 0.5, jnp.exp(s), 0.0).astype(jnp.bfloat16)
    pv_aug = jnp.dot(p, v_ref[0], preferred_element_type=jnp.float32)
    pv = pv_aug[:, :HEAD_DIM]
    denom = pv_aug[:, HEAD_DIM : HEAD_DIM + 1]
    dead = denom == 0.0
    oh = pv / jnp.where(dead, 1.0, denom)
    oh = jnp.where(dead, vm_ref[0], oh)
    contrib = jnp.dot(oh, wo_ref[0], preferred_element_type=jnp.float32)

    @pl.when(pl.program_id(1) == 0)
    def _init():
        o_ref[...] = contrib + bo_ref[...]

    @pl.when(pl.program_id(1) != 0)
    def _acc():
        o_ref[...] += contrib


@jax.jit
def kernel(query, key, value, Wq, bq, Wk, bk, Wv, bv, Wo, bo, R):
    assert query.shape[1] == 1

    r2 = R[0].transpose(0, 2, 1).reshape(HEAD_DIM, 2 * N_HASHES)
    rfull = jnp.concatenate([r2, -r2], axis=1)

    kh, sk, vaug, vmean = pl.pallas_call(
        _kv_kernel,
        grid=(HEADS,),
        in_specs=[
            pl.BlockSpec((SEQ, 1, EMBED), lambda h: (0, 0, 0)),
            pl.BlockSpec((HEAD_DIM, EMBED), lambda h: (h, 0)),
            pl.BlockSpec((1, 1, HEAD_DIM), lambda h: (h, 0, 0)),
            pl.BlockSpec((HEAD_DIM, SIG), lambda h: (0, 0)),
            pl.BlockSpec((SEQ, 1, EMBED), lambda h: (0, 0, 0)),
            pl.BlockSpec((HEAD_DIM, EMBED), lambda h: (h, 0)),
            pl.BlockSpec((1, 1, HEAD_DIM), lambda h: (h, 0, 0)),
        ],
        out_specs=[
            pl.BlockSpec((1, SEQ, HEAD_DIM), lambda h: (h, 0, 0)),
            pl.BlockSpec((1, SEQ, SIG), lambda h: (h, 0, 0)),
            pl.BlockSpec((1, SEQ, 128), lambda h: (h, 0, 0)),
            pl.BlockSpec((1, 1, HEAD_DIM), lambda h: (h, 0, 0)),
        ],
        out_shape=[
            jax.ShapeDtypeStruct((HEADS, SEQ, HEAD_DIM), jnp.bfloat16),
            jax.ShapeDtypeStruct((HEADS, SEQ, SIG), jnp.bfloat16),
            jax.ShapeDtypeStruct((HEADS, SEQ, 128), jnp.bfloat16),
            jax.ShapeDtypeStruct((HEADS, 1, HEAD_DIM), jnp.float32),
        ],
    )(
        key,
        Wk,
        bk.reshape(HEADS, 1, HEAD_DIM),
        rfull,
        value,
        Wv,
        bv.reshape(HEADS, 1, HEAD_DIM),
    )

    BQ = 1024
    NQ = SEQ // BQ
    out = pl.pallas_call(
        _attn_kernel,
        grid=(NQ, HEADS),
        in_specs=[
            pl.BlockSpec((BQ, 1, EMBED), lambda i, h: (i, 0, 0)),
            pl.BlockSpec((HEAD_DIM, EMBED), lambda i, h: (h, 0)),
            pl.BlockSpec((1, 1, HEAD_DIM), lambda i, h: (h, 0, 0)),
            pl.BlockSpec((HEAD_DIM, SIG), lambda i, h: (0, 0)),
            pl.BlockSpec((1, SEQ, HEAD_DIM), lambda i, h: (h, 0, 0)),
            pl.BlockSpec((1, SEQ, 128), lambda i, h: (h, 0, 0)),
            pl.BlockSpec((1, 1, HEAD_DIM), lambda i, h: (h, 0, 0)),
            pl.BlockSpec((1, SEQ, SIG), lambda i, h: (h, 0, 0)),
            pl.BlockSpec((1, HEAD_DIM, EMBED), lambda i, h: (h, 0, 0)),
            pl.BlockSpec((1, EMBED), lambda i, h: (0, 0)),
        ],
        out_specs=pl.BlockSpec((BQ, EMBED), lambda i, h: (i, 0)),
        out_shape=jax.ShapeDtypeStruct((SEQ, EMBED), jnp.float32),
    )(
        query,
        Wq,
        bq.reshape(HEADS, 1, HEAD_DIM),
        rfull,
        kh,
        vaug,
        vmean,
        sk,
        Wo.T.reshape(HEADS, HEAD_DIM, EMBED),
        bo[None, :],
    )

    return out[:, None, :]


# speedup vs baseline: 4.4427x; 1.0317x over previous
import jax
import jax.numpy as jnp
from jax.experimental import pallas as pl

EMBED = 1024
HEADS = 16
HEAD_DIM = EMBED // HEADS
N_HASHES = 32
SIG = 4 * N_HASHES
SEQ = 2048
CDIMS = (((1,), (1,)), ((), ()))


def _signature(h, r2):
    a = jnp.dot(h, r2, preferred_element_type=jnp.float32)
    c0 = a[:, :N_HASHES]
    c1 = a[:, N_HASHES:]
    m = jnp.maximum(jnp.abs(c0), jnp.abs(c1))
    nm = -m
    return jnp.concatenate(
        [
            (c0 == m).astype(jnp.bfloat16),
            (c1 == m).astype(jnp.bfloat16),
            (c0 == nm).astype(jnp.bfloat16),
            (c1 == nm).astype(jnp.bfloat16),
        ],
        axis=1,
    )


def _proj(x_ref, w_ref, b_ref):
    return (
        jax.lax.dot_general(
            x_ref[:, 0, :], w_ref[...], CDIMS, preferred_element_type=jnp.float32
        )
        + b_ref[0]
    )


def _kv_kernel(
    xk_ref, wk_ref, bk_ref, r_ref, xv_ref, wv_ref, bv_ref, k_ref, sk_ref, v_ref, m_ref
):
    k = _proj(xk_ref, wk_ref, bk_ref)
    k_ref[0] = k.astype(jnp.bfloat16)
    sk_ref[0] = _signature(k, r_ref[...])
    v = _proj(xv_ref, wv_ref, bv_ref)
    unit = (
        jax.lax.broadcasted_iota(jnp.int32, (SEQ, 128 - HEAD_DIM), 1) == 0
    ).astype(jnp.float32)
    v_ref[0] = jnp.concatenate([v, unit], axis=1).astype(jnp.bfloat16)
    m_ref[0] = jnp.mean(v, axis=0, keepdims=True)


def _attn_kernel(
    xq_ref, wq_ref, bq_ref, r_ref, k_ref, v_ref, vm_ref, sk_ref, wo_ref, bo_ref, o_ref
):
    q = _proj(xq_ref, wq_ref, bq_ref) * (1.0 / (HEAD_DIM ** 0.5))
    sq = _signature(q, r_ref[...])
    s = jax.lax.dot_general(
        q.astype(jnp.bfloat16), k_ref[0], CDIMS, preferred_element_type=jnp.float32
    )
    cnt = jax.lax.dot_general(sq, sk_ref[0], CDIMS, preferred_element_type=jnp.float32)
    p = jnp.where(cnt > 0.5, jnp.exp(s), 0.0).astype(jnp.bfloat16)
    pv_aug = jnp.dot(p, v_ref[0], preferred_element_type=jnp.float32)
    pv = pv_aug[:, :HEAD_DIM]
    denom = pv_aug[:, HEAD_DIM : HEAD_DIM + 1]
    dead = denom == 0.0
    oh = pv / jnp.where(dead, 1.0, denom)
    oh = jnp.where(dead, vm_ref[0], oh)
    contrib = jnp.dot(oh, wo_ref[0], preferred_element_type=jnp.float32)

    @pl.when(pl.program_id(1) == 0)
    def _init():
        o_ref[...] = contrib + bo_ref[...]

    @pl.when(pl.program_id(1) != 0)
    def _acc():
        o_ref[...] += contrib


@jax.jit
def kernel(query, key, value, Wq, bq, Wk, bk, Wv, bv, Wo, bo, R):
    assert query.shape[1] == 1

    r2 = R[0].transpose(0, 2, 1).reshape(HEAD_DIM, 2 * N_HASHES)

    kh, sk, vaug, vmean = pl.pallas_call(
        _kv_kernel,
        grid=(HEADS,),
        in_specs=[
            pl.BlockSpec((SEQ, 1, EMBED), lambda h: (0, 0, 0)),
            pl.BlockSpec((HEAD_DIM, EMBED), lambda h: (h, 0)),
            pl.BlockSpec((1, 1, HEAD_DIM), lambda h: (h, 0, 0)),
            pl.BlockSpec((HEAD_DIM, 2 * N_HASHES), lambda h: (0, 0)),
            pl.BlockSpec((SEQ, 1, EMBED), lambda h: (0, 0, 0)),
            pl.BlockSpec((HEAD_DIM, EMBED), lambda h: (h, 0)),
            pl.BlockSpec((1, 1, HEAD_DIM), lambda h: (h, 0, 0)),
        ],
        out_specs=[
            pl.BlockSpec((1, SEQ, HEAD_DIM), lambda h: (h, 0, 0)),
            pl.BlockSpec((1, SEQ, SIG), lambda h: (h, 0, 0)),
            pl.BlockSpec((1, SEQ, 128), lambda h: (h, 0, 0)),
            pl.BlockSpec((1, 1, HEAD_DIM), lambda h: (h, 0, 0)),
        ],
        out_shape=[
            jax.ShapeDtypeStruct((HEADS, SEQ, HEAD_DIM), jnp.bfloat16),
            jax.ShapeDtypeStruct((HEADS, SEQ, SIG), jnp.bfloat16),
            jax.ShapeDtypeStruct((HEADS, SEQ, 128), jnp.bfloat16),
            jax.ShapeDtypeStruct((HEADS, 1, HEAD_DIM), jnp.float32),
        ],
    )(
        key,
        Wk,
        bk.reshape(HEADS, 1, HEAD_DIM),
        r2,
        value,
        Wv,
        bv.reshape(HEADS, 1, HEAD_DIM),
    )

    BQ = 1024
    NQ = SEQ // BQ
    out = pl.pallas_call(
        _attn_kernel,
        grid=(NQ, HEADS),
        in_specs=[
            pl.BlockSpec((BQ, 1, EMBED), lambda i, h: (i, 0, 0)),
            pl.BlockSpec((HEAD_DIM, EMBED), lambda i, h: (h, 0)),
            pl.BlockSpec((1, 1, HEAD_DIM), lambda i, h: (h, 0, 0)),
            pl.BlockSpec((HEAD_DIM, 2 * N_HASHES), lambda i, h: (0, 0)),
            pl.BlockSpec((1, SEQ, HEAD_DIM), lambda i, h: (h, 0, 0)),
            pl.BlockSpec((1, SEQ, 128), lambda i, h: (h, 0, 0)),
            pl.BlockSpec((1, 1, HEAD_DIM), lambda i, h: (h, 0, 0)),
            pl.BlockSpec((1, SEQ, SIG), lambda i, h: (h, 0, 0)),
            pl.BlockSpec((1, HEAD_DIM, EMBED), lambda i, h: (h, 0, 0)),
            pl.BlockSpec((1, EMBED), lambda i, h: (0, 0)),
        ],
        out_specs=pl.BlockSpec((BQ, EMBED), lambda i, h: (i, 0)),
        out_shape=jax.ShapeDtypeStruct((SEQ, EMBED), jnp.float32),
    )(
        query,
        Wq,
        bq.reshape(HEADS, 1, HEAD_DIM),
        r2,
        kh,
        vaug,
        vmean,
        sk,
        Wo.T.reshape(HEADS, HEAD_DIM, EMBED),
        bo[None, :],
    )

    return out[:, None, :]
